# Initial kernel scaffold; baseline (speedup 1.0000x reference)
import jax
import jax.numpy as jnp
from jax.experimental import pallas as pl


def kernel(input0, input1, input2, W):
    raise NotImplementedError("write your pallas kernel here")


# speedup vs baseline: 1.4843x; 1.4843x over previous
import functools

import jax
import jax.numpy as jnp
from jax import lax
from jax.experimental import pallas as pl
from jax.experimental.pallas import tpu as pltpu
from jax.experimental.pallas import tpu_sc as plsc

VOCAB = 32320
DIM = 1024
B = 4
S = 2048

_INFO = plsc.get_sparse_core_info()
_NC, _NS = _INFO.num_cores, _INFO.num_subcores
_NW = _NC * _NS
_N_IDX = B * S
_PER_W = _N_IDX // _NW
_CHUNK = 64
_NCHUNK = _PER_W // _CHUNK


@functools.partial(
    pl.kernel,
    out_type=jax.ShapeDtypeStruct((_N_IDX, DIM), jnp.float32),
    mesh=plsc.VectorSubcoreMesh(core_axis_name="c", subcore_axis_name="s"),
    scratch_types=[
        pltpu.VMEM((_NCHUNK, _CHUNK), jnp.int32),
        pltpu.VMEM((_CHUNK, DIM), jnp.float32),
        pltpu.SemaphoreType.DMA,
    ],
)
def _gather_rows(idx_hbm, table_hbm, out_hbm, idx_v, rows_v, sem):
    wid = lax.axis_index("s") * _NC + lax.axis_index("c")
    base = wid * _PER_W
    pltpu.sync_copy(idx_hbm.at[wid], idx_v)
    for c in range(_NCHUNK):
        pltpu.async_copy(table_hbm.at[idx_v.at[c]], rows_v, sem).wait()
        pltpu.sync_copy(rows_v, out_hbm.at[pl.ds(base + c * _CHUNK, _CHUNK)])


def kernel(input0, input1, input2, W):
    idx = input0.reshape(_NW, _NCHUNK, _CHUNK).astype(jnp.int32)
    rows = _gather_rows(idx, W)
    return (input1, input2, rows.reshape(B, S, DIM))


# speedup vs baseline: 1.5205x; 1.0244x over previous
import functools

import jax
import jax.numpy as jnp
from jax import lax
from jax.experimental import pallas as pl
from jax.experimental.pallas import tpu as pltpu
from jax.experimental.pallas import tpu_sc as plsc

VOCAB = 32320
DIM = 1024
B = 4
S = 2048

_INFO = plsc.get_sparse_core_info()
_NC, _NS = _INFO.num_cores, _INFO.num_subcores
_NW = _NC * _NS
_N_IDX = B * S
_PER_W = _N_IDX // _NW
_CHUNK = 32
_NCHUNK = _PER_W // _CHUNK


@functools.partial(
    pl.kernel,
    out_type=jax.ShapeDtypeStruct((_N_IDX, DIM), jnp.float32),
    mesh=plsc.VectorSubcoreMesh(core_axis_name="c", subcore_axis_name="s"),
    scratch_types=[
        pltpu.VMEM((_NCHUNK, _CHUNK), jnp.int32),
        pltpu.VMEM((_CHUNK, DIM), jnp.float32),
        pltpu.VMEM((_CHUNK, DIM), jnp.float32),
        pltpu.SemaphoreType.DMA,
        pltpu.SemaphoreType.DMA,
        pltpu.SemaphoreType.DMA,
        pltpu.SemaphoreType.DMA,
    ],
)
def _gather_rows(idx_hbm, table_hbm, out_hbm, idx_v, rows0, rows1,
                 g0, g1, s0, s1):
    wid = lax.axis_index("s") * _NC + lax.axis_index("c")
    base = wid * _PER_W
    bufs, gsems, ssems = (rows0, rows1), (g0, g1), (s0, s1)

    pltpu.sync_copy(idx_hbm.at[wid], idx_v)

    gathers = [None, None]
    stores = [None, None]
    gathers[0] = pltpu.async_copy(table_hbm.at[idx_v.at[0]], bufs[0], gsems[0])
    for c in range(_NCHUNK):
        cur, nxt = c & 1, (c + 1) & 1
        if c + 1 < _NCHUNK:
            if c + 1 >= 2:
                stores[nxt].wait()
            gathers[nxt] = pltpu.async_copy(
                table_hbm.at[idx_v.at[c + 1]], bufs[nxt], gsems[nxt])
        gathers[cur].wait()
        stores[cur] = pltpu.async_copy(
            bufs[cur], out_hbm.at[pl.ds(base + c * _CHUNK, _CHUNK)],
            ssems[cur])
    stores[0].wait()
    stores[1].wait()


def kernel(input0, input1, input2, W):
    idx = input0.reshape(_NW, _NCHUNK, _CHUNK).astype(jnp.int32)
    rows = _gather_rows(idx, W)
    return (input1, input2, rows.reshape(B, S, DIM))


# speedup vs baseline: 1.5339x; 1.0088x over previous
import functools

import jax
import jax.numpy as jnp
from jax import lax
from jax.experimental import pallas as pl
from jax.experimental.pallas import tpu as pltpu
from jax.experimental.pallas import tpu_sc as plsc

VOCAB = 32320
DIM = 1024
B = 4
S = 2048

_INFO = plsc.get_sparse_core_info()
_NC, _NS = _INFO.num_cores, _INFO.num_subcores
_NW = _NC * _NS
_N_IDX = B * S
_PER_W = _N_IDX // _NW
_W_PER_ROW = S // _PER_W
_CHUNK = 16
_NCHUNK = _PER_W // _CHUNK
_NBUF = 4


@functools.partial(
    pl.kernel,
    out_type=jax.ShapeDtypeStruct((_N_IDX, DIM), jnp.float32),
    mesh=plsc.VectorSubcoreMesh(core_axis_name="c", subcore_axis_name="s"),
    scratch_types=(
        [pltpu.VMEM((_PER_W,), jnp.int32)]
        + [pltpu.VMEM((_CHUNK, DIM), jnp.float32)] * _NBUF
        + [pltpu.SemaphoreType.DMA] * (2 * _NBUF)
    ),
)
def _gather_rows(idx_hbm, table_hbm, out_hbm, idx_v, *bufs_and_sems):
    bufs = bufs_and_sems[:_NBUF]
    gsems = bufs_and_sems[_NBUF:2 * _NBUF]
    ssems = bufs_and_sems[2 * _NBUF:]
    wid = lax.axis_index("s") * _NC + lax.axis_index("c")
    base = wid * _PER_W
    row = wid // _W_PER_ROW
    col = (wid % _W_PER_ROW) * _PER_W

    pltpu.sync_copy(idx_hbm.at[row, pl.ds(col, _PER_W)], idx_v)

    gathers = [None] * _NBUF
    stores = [None] * _NBUF
    for c in range(min(_NBUF, _NCHUNK)):
        gathers[c] = pltpu.async_copy(
            table_hbm.at[idx_v.at[pl.ds(c * _CHUNK, _CHUNK)]], bufs[c],
            gsems[c])
    for c in range(_NCHUNK):
        b = c % _NBUF
        gathers[b].wait()
        stores[b] = pltpu.async_copy(
            bufs[b], out_hbm.at[pl.ds(base + c * _CHUNK, _CHUNK)], ssems[b])
        nc = c + _NBUF
        if nc < _NCHUNK:
            stores[b].wait()
            gathers[b] = pltpu.async_copy(
                table_hbm.at[idx_v.at[pl.ds(nc * _CHUNK, _CHUNK)]], bufs[b],
                gsems[b])
    for c in range(max(0, _NCHUNK - _NBUF), _NCHUNK):
        stores[c % _NBUF].wait()


def kernel(input0, input1, input2, W):
    idx = input0.astype(jnp.int32)
    rows = _gather_rows(idx, W)
    return (input1, input2, rows.reshape(B, S, DIM))
